# Initial kernel scaffold; baseline (speedup 1.0000x reference)
import jax
import jax.numpy as jnp
from jax.experimental import pallas as pl


def kernel(b0_w, b0_scale, b0_offset, dil_w, dil_scale, dil_offset, pool_w, pool_scale, pool_offset, proj_w, proj_scale, proj_offset, head_w, head_scale, head_offset, cls_w, cls_b, x):
    raise NotImplementedError("write your pallas kernel here")


# speedup vs baseline: 1.5713x; 1.5713x over previous
import functools

import jax
import jax.numpy as jnp
from jax.experimental import pallas as pl
from jax.experimental.pallas import tpu as pltpu

_DILATIONS = (12, 24, 36)


def _fused_kernel(xp_ref, b0w_ref, b0o_ref, dilw_ref, dilo_ref,
                  poolw_ref, poolo_ref, projw_ref, projo_ref,
                  headw_ref, heado_ref, clsw_ref, clsb_ref,
                  o_ref, pacc, conv, pbuf, *, H, W, P, dils):
    cin = xp_ref.shape[-1]
    C = b0w_ref.shape[-1]
    CP = clsw_ref.shape[-1]
    HW = H * W
    f32 = jnp.float32
    bf16 = jnp.bfloat16

    interior = xp_ref[0, P:P + H, P:P + W, :].reshape(HW, cin)

    mean = jnp.mean(interior.astype(f32), axis=0, keepdims=True)
    pooled = jnp.dot(mean.astype(bf16), poolw_ref[...],
                     preferred_element_type=f32)
    pooled = jnp.maximum(pooled + poolo_ref[...], 0.0)
    pool_proj = jnp.dot(pooled.astype(bf16), projw_ref[4],
                        preferred_element_type=f32)

    b0 = jnp.dot(interior, b0w_ref[...], preferred_element_type=f32)
    b0 = jnp.maximum(b0 + b0o_ref[...], 0.0)
    pacc[...] = (jnp.dot(b0.astype(bf16), projw_ref[0],
                         preferred_element_type=f32) + pool_proj)

    for i, d in enumerate(dils):
        first = True
        for kh in (1, 0, 2):
            dh = (kh - 1) * d
            lo = max(0, -dh)
            hi = H - max(0, dh)
            if lo >= hi:
                continue
            for kw in range(3):
                dw = (kw - 1) * d
                patch = xp_ref[0, P + dh + lo:P + dh + hi,
                               P + dw:P + dw + W, :].reshape((hi - lo) * W, cin)
                contrib = jnp.dot(patch, dilw_ref[i * 9 + kh * 3 + kw],
                                  preferred_element_type=f32)
                if first:
                    conv[...] = contrib
                    first = False
                else:
                    conv[lo * W:hi * W, :] += contrib
        bi = jnp.maximum(conv[...] + dilo_ref[i], 0.0)
        pacc[...] += jnp.dot(bi.astype(bf16), projw_ref[i + 1],
                             preferred_element_type=f32)

    proj = jnp.maximum(pacc[...] + projo_ref[...], 0.0).astype(bf16)
    pbuf[0:1, :, :] = jnp.zeros((1, W + 2, C), bf16)
    pbuf[H + 1:H + 2, :, :] = jnp.zeros((1, W + 2, C), bf16)
    pbuf[:, 0:1, :] = jnp.zeros((H + 2, 1, C), bf16)
    pbuf[:, W + 1:W + 2, :] = jnp.zeros((H + 2, 1, C), bf16)
    pbuf[1:H + 1, 1:W + 1, :] = proj.reshape(H, W, C)

    for kh in range(3):
        for kw in range(3):
            patch = pbuf[kh:kh + H, kw:kw + W, :].reshape(HW, C)
            contrib = jnp.dot(patch, headw_ref[kh * 3 + kw],
                              preferred_element_type=f32)
            if kh == 0 and kw == 0:
                conv[...] = contrib
            else:
                conv[...] += contrib
    h = jnp.maximum(conv[...] + heado_ref[...], 0.0).astype(bf16)

    logits = jax.lax.dot_general(clsw_ref[...], h,
                                 (((0,), (1,)), ((), ())),
                                 preferred_element_type=f32)
    o_ref[...] = (logits + clsb_ref[...]).reshape(1, CP, HW)


def kernel(b0_w, b0_scale, b0_offset, dil_w, dil_scale, dil_offset,
           pool_w, pool_scale, pool_offset, proj_w, proj_scale, proj_offset,
           head_w, head_scale, head_offset, cls_w, cls_b, x):
    N, cin, H, W = x.shape
    C = b0_w.shape[-1]
    P = max(_DILATIONS)
    nc = cls_w.shape[1]
    CP = max(32, ((nc + 7) // 8) * 8)
    HW = H * W
    bf = jnp.bfloat16

    xh = jnp.transpose(x, (0, 2, 3, 1)).astype(bf)
    xp = jnp.pad(xh, ((0, 0), (P, P), (P, P), (0, 0)))
    Hp, Wp = H + 2 * P, W + 2 * P

    b0w = (b0_w * b0_scale).astype(bf)
    dilw = (dil_w * dil_scale[:, None, None]).reshape(9 * len(_DILATIONS),
                                                     cin, C).astype(bf)
    poolw = (pool_w * pool_scale).astype(bf)
    projw = (proj_w * proj_scale).astype(bf)
    headw = (head_w * head_scale).reshape(9, C, C).astype(bf)
    clsw = jnp.pad(cls_w, ((0, 0), (0, CP - nc))).astype(bf)
    clsb = jnp.pad(cls_b, ((0, 0), (0, CP - nc))).reshape(CP, 1)

    def const(*shape):
        nd = len(shape)
        return pl.BlockSpec(shape, lambda n, _nd=nd: (0,) * _nd)

    out = pl.pallas_call(
        functools.partial(_fused_kernel, H=H, W=W, P=P, dils=_DILATIONS),
        out_shape=jax.ShapeDtypeStruct((N, CP, HW), jnp.float32),
        grid=(N,),
        in_specs=[
            pl.BlockSpec((1, Hp, Wp, cin), lambda n: (n, 0, 0, 0)),
            const(cin, C), const(1, C),
            const(9 * len(_DILATIONS), cin, C), const(len(_DILATIONS), 1, C),
            const(cin, C), const(1, C),
            const(5, C, C), const(1, C),
            const(9, C, C), const(1, C),
            const(cin, CP), const(CP, 1),
        ],
        out_specs=pl.BlockSpec((1, CP, HW), lambda n: (n, 0, 0)),
        scratch_shapes=[
            pltpu.VMEM((HW, C), jnp.float32),
            pltpu.VMEM((HW, C), jnp.float32),
            pltpu.VMEM((H + 2, W + 2, C), bf),
        ],
        compiler_params=pltpu.CompilerParams(
            dimension_semantics=("parallel",),
            vmem_limit_bytes=110 * 1024 * 1024),
    )(xp, b0w, b0_offset, dilw, dil_offset, poolw, pool_offset,
      projw, proj_offset, headw, head_offset, clsw, clsb)
    return out[:, :nc, :].reshape(N, nc, H, W)
